# baseline (speedup 1.0000x reference)
import functools

import jax
import jax.numpy as jnp
from jax import lax
from jax.experimental import pallas as pl
from jax.experimental.pallas import tpu as pltpu
from jax.experimental.pallas import tpu_sc as plsc

_NW = 32
_CHUNK = 128
_M = 4
_TCOLS = 16384
_SEG = 8192


@functools.lru_cache(maxsize=None)
def _build_gather(total_rows: int, embed: int):
    rows_per_w = total_rows // _NW
    n_chunks = rows_per_w // _CHUNK
    n_groups = n_chunks // _M
    rpk = 128 // embed
    sub = _SEG // rpk
    cps = sub // _CHUNK
    assert n_chunks % _M == 0 and n_groups >= 4 and (n_groups - 2) % 2 == 0

    mesh = plsc.VectorSubcoreMesh(core_axis_name="c", subcore_axis_name="s")

    @functools.partial(
        pl.kernel,
        out_type=jax.ShapeDtypeStruct((total_rows // rpk, rpk, embed),
                                      jnp.float32),
        mesh=mesh,
        scratch_types=[
            pltpu.VMEM((n_chunks, _CHUNK), jnp.int32),
            pltpu.VMEM((_M, _CHUNK, embed), jnp.float32),
            pltpu.VMEM((_M, _CHUNK, embed), jnp.float32),
            pltpu.SemaphoreType.DMA,
            pltpu.SemaphoreType.DMA,
        ],
        compiler_params=pltpu.CompilerParams(
            use_tc_tiling_on_sc=False, needs_layout_passes=False
        ),
    )
    def gather_kernel(table, idx, out, idx_v, buf_a, buf_b, sem_g, sem_s):
        wid = lax.axis_index("s") * 2 + lax.axis_index("c")
        chunk0 = wid * n_chunks

        pltpu.sync_copy(idx.at[pl.ds(chunk0, n_chunks)], idx_v)

        sets = (buf_a, buf_b)

        def out_slab(c):
            ch = chunk0 + c
            seg = ch // (rpk * cps)
            r = ch % (rpk * cps)
            a = r // cps
            m0 = seg * sub + (r % cps) * _CHUNK
            return out.at[pl.ds(m0, _CHUNK), a, :]

        def fire_gathers(g, dst):
            for j in range(_M):
                pltpu.async_copy(table.at[idx_v.at[g * _M + j]], dst.at[j], sem_g)

        def wait_gathers(g, dst):
            for j in range(_M):
                pltpu.make_async_copy(
                    table.at[idx_v.at[g * _M + j]], dst.at[j], sem_g
                ).wait()

        def fire_outs(g, src):
            for j in range(_M):
                pltpu.async_copy(src.at[j], out_slab(g * _M + j), sem_s)

        def wait_outs(g, src):
            for j in range(_M):
                pltpu.make_async_copy(
                    src.at[j], out_slab(g * _M + j), sem_s
                ).wait()

        fire_gathers(0, sets[0])
        wait_gathers(0, sets[0])
        fire_outs(0, sets[0])
        fire_gathers(1, sets[1])

        @pl.loop(0, (n_groups - 2) // 2)
        def _pair(hh):
            g0 = 2 * hh + 1
            for p, g in ((1, g0), (0, g0 + 1)):
                wait_gathers(g, sets[p])
                fire_outs(g, sets[p])
                wait_outs(g - 1, sets[1 - p])
                fire_gathers(g + 1, sets[1 - p])

        g_last = n_groups - 1
        wait_gathers(g_last, sets[1])
        fire_outs(g_last, sets[1])
        wait_outs(g_last - 1, sets[0])
        wait_outs(g_last, sets[1])

    return gather_kernel


@functools.lru_cache(maxsize=None)
def _build_transpose(vocab: int, embed: int):
    rpk = 128 // embed
    okr = _TCOLS // rpk
    n_blocks = pl.cdiv(vocab, _TCOLS)

    def body(i_ref, o_ref):
        t = i_ref[...].T
        for a in range(rpk):
            o_ref[:, a * embed:(a + 1) * embed] = t[a * okr:(a + 1) * okr, :]

    return pl.pallas_call(
        body,
        grid=(n_blocks,),
        in_specs=[pl.BlockSpec((embed, _TCOLS), lambda i: (0, i))],
        out_specs=pl.BlockSpec((okr, 128), lambda i: (i, 0)),
        out_shape=jax.ShapeDtypeStruct((n_blocks * okr, 128), jnp.float32),
    )


def _remap_indices(idx, embed):
    rpk = 128 // embed
    okr = _TCOLS // rpk
    blk = idx // _TCOLS
    m = idx % _TCOLS
    return (blk * okr + m % okr) * rpk + m // okr


@functools.lru_cache(maxsize=None)
def _build_out_transpose(batch: int, hist: int, embed: int):
    rpk = 128 // embed
    seg_rows = _SEG // rpk
    sub = _SEG // rpk
    n_seg = batch // _SEG

    def body(i_ref, o_ref):
        x = i_ref[...]
        for a in range(rpk):
            o_ref[0, :, a * sub:(a + 1) * sub] = x[:, a * embed:(a + 1) * embed].T

    return pl.pallas_call(
        body,
        grid=(hist, n_seg),
        in_specs=[
            pl.BlockSpec((seg_rows, 128), lambda h, j: (h * n_seg + j, 0))
        ],
        out_specs=pl.BlockSpec((1, embed, _SEG), lambda h, j: (h, 0, j)),
        out_shape=jax.ShapeDtypeStruct((hist, embed, batch), jnp.float32),
    )


def kernel(emb_mat, indices):
    vocab, embed = emb_mat.shape
    batch, hist = indices.shape
    total = indices.size
    idx = _remap_indices(indices.T.reshape(total).astype(jnp.int32), embed)
    idx2d = idx.reshape(total // _CHUNK, _CHUNK)
    table_r = _build_transpose(vocab, embed)(emb_mat.T)
    table_flat = table_r.reshape(table_r.size // embed, embed)
    g = _build_gather(total, embed)(table_flat, idx2d)
    g2d = g.reshape(total * embed // 128, 128)
    p = _build_out_transpose(batch, hist, embed)(g2d)
    return p.transpose(2, 0, 1)


# speedup vs baseline: 1.9610x; 1.9610x over previous
import functools

import jax
import jax.numpy as jnp
from jax import lax
from jax.experimental import pallas as pl
from jax.experimental.pallas import tpu as pltpu
from jax.experimental.pallas import tpu_sc as plsc

_NW = 32
_CHUNK = 128
_M = 4
_TCOLS = 16384
_SEG = 8192


@functools.lru_cache(maxsize=None)
def _build_gather(total_rows: int, embed: int):
    rows_per_w = total_rows // _NW
    n_chunks = rows_per_w // _CHUNK
    n_groups = n_chunks // _M
    rpk = 128 // embed
    sub = _SEG // rpk
    cps = sub // _CHUNK
    assert n_chunks % _M == 0 and n_groups >= 4 and (n_groups - 2) % 2 == 0

    mesh = plsc.VectorSubcoreMesh(core_axis_name="c", subcore_axis_name="s")

    @functools.partial(
        pl.kernel,
        out_type=jax.ShapeDtypeStruct((total_rows // rpk, 128), jnp.float32),
        mesh=mesh,
        scratch_types=[
            pltpu.VMEM((n_chunks, _CHUNK), jnp.int32),
            pltpu.VMEM((_M, _CHUNK, embed), jnp.float32),
            pltpu.VMEM((_M, _CHUNK, embed), jnp.float32),
            pltpu.SemaphoreType.DMA,
            pltpu.SemaphoreType.DMA,
        ],
        compiler_params=pltpu.CompilerParams(
            use_tc_tiling_on_sc=False, needs_layout_passes=False
        ),
    )
    def gather_kernel(table, idx, out, idx_v, buf_a, buf_b, sem_g, sem_s):
        wid = lax.axis_index("s") * 2 + lax.axis_index("c")
        chunk0 = wid * n_chunks

        pltpu.sync_copy(idx.at[pl.ds(chunk0, n_chunks)], idx_v)

        sets = (buf_a, buf_b)

        def out_slab(c):
            ch = chunk0 + c
            seg = ch // (rpk * cps)
            r = ch % (rpk * cps)
            a = r // cps
            m0 = seg * sub + (r % cps) * _CHUNK
            return out.at[pl.ds(m0, _CHUNK), pl.ds(a * embed, embed)]

        def fire_gathers(g, dst):
            for j in range(_M):
                pltpu.async_copy(table.at[idx_v.at[g * _M + j]], dst.at[j], sem_g)

        def wait_gathers(g, dst):
            for j in range(_M):
                pltpu.make_async_copy(
                    table.at[idx_v.at[g * _M + j]], dst.at[j], sem_g
                ).wait()

        def fire_outs(g, src):
            for j in range(_M):
                pltpu.async_copy(src.at[j], out_slab(g * _M + j), sem_s)

        def wait_outs(g, src):
            for j in range(_M):
                pltpu.make_async_copy(
                    src.at[j], out_slab(g * _M + j), sem_s
                ).wait()

        fire_gathers(0, sets[0])
        wait_gathers(0, sets[0])
        fire_outs(0, sets[0])
        fire_gathers(1, sets[1])

        @pl.loop(0, (n_groups - 2) // 2)
        def _pair(hh):
            g0 = 2 * hh + 1
            for p, g in ((1, g0), (0, g0 + 1)):
                wait_gathers(g, sets[p])
                fire_outs(g, sets[p])
                wait_outs(g - 1, sets[1 - p])
                fire_gathers(g + 1, sets[1 - p])

        g_last = n_groups - 1
        wait_gathers(g_last, sets[1])
        fire_outs(g_last, sets[1])
        wait_outs(g_last - 1, sets[0])
        wait_outs(g_last, sets[1])

    return gather_kernel


@functools.lru_cache(maxsize=None)
def _build_transpose(vocab: int, embed: int):
    rpk = 128 // embed
    okr = _TCOLS // rpk
    n_blocks = pl.cdiv(vocab, _TCOLS)

    def body(i_ref, o_ref):
        t = i_ref[...].T
        for a in range(rpk):
            o_ref[:, a * embed:(a + 1) * embed] = t[a * okr:(a + 1) * okr, :]

    return pl.pallas_call(
        body,
        grid=(n_blocks,),
        in_specs=[pl.BlockSpec((embed, _TCOLS), lambda i: (0, i))],
        out_specs=pl.BlockSpec((okr, 128), lambda i: (i, 0)),
        out_shape=jax.ShapeDtypeStruct((n_blocks * okr, 128), jnp.float32),
    )


def _remap_indices(idx, embed):
    rpk = 128 // embed
    okr = _TCOLS // rpk
    blk = idx // _TCOLS
    m = idx % _TCOLS
    return (blk * okr + m % okr) * rpk + m // okr


@functools.lru_cache(maxsize=None)
def _build_out_transpose(batch: int, hist: int, embed: int):
    rpk = 128 // embed
    seg_rows = _SEG // rpk
    sub = _SEG // rpk
    n_seg = batch // _SEG

    def body(i_ref, o_ref):
        x = i_ref[...]
        for a in range(rpk):
            o_ref[0, :, a * sub:(a + 1) * sub] = x[:, a * embed:(a + 1) * embed].T

    return pl.pallas_call(
        body,
        grid=(hist, n_seg),
        in_specs=[
            pl.BlockSpec((seg_rows, 128), lambda h, j: (h * n_seg + j, 0))
        ],
        out_specs=pl.BlockSpec((1, embed, _SEG), lambda h, j: (h, 0, j)),
        out_shape=jax.ShapeDtypeStruct((hist, embed, batch), jnp.float32),
    )


def kernel(emb_mat, indices):
    vocab, embed = emb_mat.shape
    batch, hist = indices.shape
    total = indices.size
    idx = _remap_indices(indices.T.reshape(total).astype(jnp.int32), embed)
    idx2d = idx.reshape(total // _CHUNK, _CHUNK)
    table_r = _build_transpose(vocab, embed)(emb_mat.T)
    table_flat = table_r.reshape(table_r.size // embed, embed)
    g2d = _build_gather(total, embed)(table_flat, idx2d)
    p = _build_out_transpose(batch, hist, embed)(g2d)
    return p.transpose(2, 0, 1)
